# baseline (device time: 50657 ns/iter reference)
import jax
import jax.numpy as jnp
from jax import lax
from jax.experimental import pallas as pl
from jax.experimental.pallas import tpu as pltpu

N_DEV = 16
NP = 4
NZ = 4
MP = 64
NQ = 16


def kernel(A, B):
    m, k = A.shape
    _, n = B.shape
    qw = n // NQ

    def body(a_ref, b_ref, out_ref,
             sd, sr, sc, rd, rr, rc, ps, zrecv,
             sd_s, sd_r, sr_s, sr_r, sc_s, sc_r, zs_sems, zr_sems):
        i = lax.axis_index("i")
        z = i // NP
        p = i % NP
        zbase = z * NP
        plane_r = zbase + (p + 1) % NP
        plane_l = zbase + (p + NP - 1) % NP

        barrier = pltpu.get_barrier_semaphore()
        peers = [plane_r, plane_l] + [((z + dz) % NZ) * NP + p for dz in (1, 2, 3)]
        for tgt in peers:
            pl.semaphore_signal(
                barrier, inc=1,
                device_id=(tgt,), device_id_type=pl.DeviceIdType.MESH,
            )

        def partial(qset, zp, q):
            return jnp.dot(
                a_ref[pl.ds(zp * (NP * MP) + qset * MP, MP), :],
                b_ref[:, q * qw:(q + 1) * qw],
                preferred_element_type=jnp.float32,
            )

        def cw(q):
            return q % 2 == 0

        def qd(q):
            return (p + 1) % NP if cw(q) else (p + NP - 1) % NP

        def qr(q):
            return (p + 2) % NP

        def qc(q):
            return (p + NP - 1) % NP if cw(q) else (p + 1) % NP

        def mk(src, dst, ssem, rsem, q, tgt):
            return pltpu.make_async_remote_copy(
                src_ref=src.at[q], dst_ref=dst.at[q],
                send_sem=ssem.at[q], recv_sem=rsem.at[q],
                device_id=(tgt,), device_id_type=pl.DeviceIdType.MESH,
            )

        for q in range(NQ):
            for zp in range(NZ):
                sr[q, zp, :, :] = partial(qr(q), zp, q)

        pl.semaphore_wait(barrier, 5)

        relay = [mk(sr, rr, sr_s, sr_r, q, plane_l if cw(q) else plane_r)
                 for q in range(NQ)]
        direct = [mk(sd, rd, sd_s, sd_r, q, plane_r if cw(q) else plane_l)
                  for q in range(NQ)]
        comb = [mk(sc, rc, sc_s, sc_r, q, plane_l if cw(q) else plane_r)
                for q in range(NQ)]

        for q in range(NQ):
            relay[q].start()

        for pair in tuple((2 * j, 2 * j + 1) for j in range(NQ // 2)):
            for q in pair:
                for zp in range(NZ):
                    sc[q, zp, :, :] = partial(qc(q), zp, q)
            for q in pair:
                relay[q].wait_recv()
                for zp in range(NZ):
                    sc[q, zp, :, :] = sc[q, zp, :, :] + rr[q, zp, :, :]
                comb[q].start()
            for q in pair:
                for zp in range(NZ):
                    sd[q, zp, :, :] = partial(qd(q), zp, q)
                direct[q].start()

        for q in range(NQ):
            for zp in range(NZ):
                ps[q, zp, :, :] = partial(p, zp, q)

        zsends = []
        for q in range(NQ):
            direct[q].wait_recv()
            comb[q].wait_recv()
            for zp in range(NZ):
                ps[q, zp, :, :] = ps[q, zp, :, :] + rd[q, zp, :, :] + rc[q, zp, :, :]
            for dz in (1, 2, 3):
                zt = (z + dz) % NZ
                s = pltpu.make_async_remote_copy(
                    src_ref=ps.at[q, zt], dst_ref=zrecv.at[q, z],
                    send_sem=zs_sems.at[q, dz - 1], recv_sem=zr_sems.at[q, z],
                    device_id=(zt * NP + p,),
                    device_id_type=pl.DeviceIdType.MESH,
                )
                s.start()
                zsends.append(s)

        for q in range(NQ):
            acc = ps[q, z, :, :]
            for dz in (1, 2, 3):
                zs = (z + dz) % NZ
                rwait = pltpu.make_async_remote_copy(
                    src_ref=ps.at[q, 0], dst_ref=zrecv.at[q, zs],
                    send_sem=zs_sems.at[q, dz - 1], recv_sem=zr_sems.at[q, zs],
                    device_id=(i,), device_id_type=pl.DeviceIdType.MESH,
                )
                rwait.wait_recv()
                acc = acc + zrecv[q, zs, :, :]
            out_ref[:, q * qw:(q + 1) * qw] = acc

        for q in range(NQ):
            relay[q].wait_send()
            direct[q].wait_send()
            comb[q].wait_send()
        for s in zsends:
            s.wait_send()

    buf = pltpu.VMEM((NQ, NZ, MP, qw), jnp.float32)
    return pl.pallas_call(
        body,
        out_shape=jax.ShapeDtypeStruct((MP, n), jnp.float32),
        in_specs=[
            pl.BlockSpec(memory_space=pltpu.VMEM),
            pl.BlockSpec(memory_space=pltpu.VMEM),
        ],
        out_specs=pl.BlockSpec(memory_space=pltpu.VMEM),
        scratch_shapes=[
            buf, buf, buf, buf, buf, buf, buf,
            pltpu.VMEM((NQ, NZ, MP, qw), jnp.float32),
            pltpu.SemaphoreType.DMA((NQ,)),
            pltpu.SemaphoreType.DMA((NQ,)),
            pltpu.SemaphoreType.DMA((NQ,)),
            pltpu.SemaphoreType.DMA((NQ,)),
            pltpu.SemaphoreType.DMA((NQ,)),
            pltpu.SemaphoreType.DMA((NQ,)),
            pltpu.SemaphoreType.DMA((NQ, NZ - 1)),
            pltpu.SemaphoreType.DMA((NQ, NZ)),
        ],
        compiler_params=pltpu.CompilerParams(collective_id=0),
    )(A, B)


# device time: 31498 ns/iter; 1.6083x vs baseline; 1.6083x over previous
import jax
import jax.numpy as jnp
from jax import lax
from jax.experimental import pallas as pl
from jax.experimental.pallas import tpu as pltpu

N_DEV = 16
NP = 4
NZ = 4
MP = 64
NQ = 8


def kernel(A, B):
    m, k = A.shape
    _, n = B.shape
    qw = n // NQ

    def body(a_ref, b_ref, out_ref,
             sd, sr, sc, rd, rr, rc, ps, zrecv,
             sd_s, sd_r, sr_s, sr_r, sc_s, sc_r, zs_sems, zr_sems):
        i = lax.axis_index("i")
        z = i // NP
        p = i % NP
        zbase = z * NP
        plane_r = zbase + (p + 1) % NP
        plane_l = zbase + (p + NP - 1) % NP

        barrier = pltpu.get_barrier_semaphore()
        peers = [plane_r, plane_l] + [((z + dz) % NZ) * NP + p for dz in (1, 2, 3)]
        for tgt in peers:
            pl.semaphore_signal(
                barrier, inc=1,
                device_id=(tgt,), device_id_type=pl.DeviceIdType.MESH,
            )

        def partial(qset, zp, q):
            return jnp.dot(
                a_ref[pl.ds(zp * (NP * MP) + qset * MP, MP), :],
                b_ref[:, q * qw:(q + 1) * qw],
                preferred_element_type=jnp.float32,
            )

        def cw(q):
            return q % 2 == 0

        def qd(q):
            return (p + 1) % NP if cw(q) else (p + NP - 1) % NP

        def qr(q):
            return (p + 2) % NP

        def qc(q):
            return (p + NP - 1) % NP if cw(q) else (p + 1) % NP

        def mk(src, dst, ssem, rsem, q, tgt):
            return pltpu.make_async_remote_copy(
                src_ref=src.at[q], dst_ref=dst.at[q],
                send_sem=ssem.at[q], recv_sem=rsem.at[q],
                device_id=(tgt,), device_id_type=pl.DeviceIdType.MESH,
            )

        for q in range(NQ):
            for zp in range(NZ):
                sr[q, zp, :, :] = partial(qr(q), zp, q)

        pl.semaphore_wait(barrier, 5)

        relay = [mk(sr, rr, sr_s, sr_r, q, plane_l if cw(q) else plane_r)
                 for q in range(NQ)]
        direct = [mk(sd, rd, sd_s, sd_r, q, plane_r if cw(q) else plane_l)
                  for q in range(NQ)]
        comb = [mk(sc, rc, sc_s, sc_r, q, plane_l if cw(q) else plane_r)
                for q in range(NQ)]

        for q in range(NQ):
            relay[q].start()

        for pair in tuple((2 * j, 2 * j + 1) for j in range(NQ // 2)):
            for q in pair:
                for zp in range(NZ):
                    sc[q, zp, :, :] = partial(qc(q), zp, q)
            for q in pair:
                relay[q].wait_recv()
                for zp in range(NZ):
                    sc[q, zp, :, :] = sc[q, zp, :, :] + rr[q, zp, :, :]
                comb[q].start()
            for q in pair:
                for zp in range(NZ):
                    sd[q, zp, :, :] = partial(qd(q), zp, q)
                direct[q].start()

        for q in range(NQ):
            for zp in range(NZ):
                ps[q, zp, :, :] = partial(p, zp, q)

        zsends = []
        for j in range(NQ // 2):
            for q in (2 * j, 2 * j + 1):
                direct[q].wait_recv()
                comb[q].wait_recv()
                for zp in range(NZ):
                    ps[q, zp, :, :] = (
                        ps[q, zp, :, :] + rd[q, zp, :, :] + rc[q, zp, :, :])
            for dz in (1, 2, 3):
                zt = (z + dz) % NZ
                s = pltpu.make_async_remote_copy(
                    src_ref=ps.at[pl.ds(2 * j, 2), zt],
                    dst_ref=zrecv.at[pl.ds(2 * j, 2), z],
                    send_sem=zs_sems.at[j, dz - 1], recv_sem=zr_sems.at[j, z],
                    device_id=(zt * NP + p,),
                    device_id_type=pl.DeviceIdType.MESH,
                )
                s.start()
                zsends.append(s)

        for j in range(NQ // 2):
            for dz in (1, 2, 3):
                zs = (z + dz) % NZ
                rwait = pltpu.make_async_remote_copy(
                    src_ref=ps.at[pl.ds(2 * j, 2), 0],
                    dst_ref=zrecv.at[pl.ds(2 * j, 2), zs],
                    send_sem=zs_sems.at[j, dz - 1], recv_sem=zr_sems.at[j, zs],
                    device_id=(i,), device_id_type=pl.DeviceIdType.MESH,
                )
                rwait.wait_recv()
            for q in (2 * j, 2 * j + 1):
                acc = ps[q, z, :, :]
                for dz in (1, 2, 3):
                    acc = acc + zrecv[q, (z + dz) % NZ, :, :]
                out_ref[:, q * qw:(q + 1) * qw] = acc

        for q in range(NQ):
            relay[q].wait_send()
            direct[q].wait_send()
            comb[q].wait_send()
        for s in zsends:
            s.wait_send()

    buf = pltpu.VMEM((NQ, NZ, MP, qw), jnp.float32)
    return pl.pallas_call(
        body,
        out_shape=jax.ShapeDtypeStruct((MP, n), jnp.float32),
        in_specs=[
            pl.BlockSpec(memory_space=pltpu.VMEM),
            pl.BlockSpec(memory_space=pltpu.VMEM),
        ],
        out_specs=pl.BlockSpec(memory_space=pltpu.VMEM),
        scratch_shapes=[
            buf, buf, buf, buf, buf, buf, buf,
            pltpu.VMEM((NQ, NZ, MP, qw), jnp.float32),
            pltpu.SemaphoreType.DMA((NQ,)),
            pltpu.SemaphoreType.DMA((NQ,)),
            pltpu.SemaphoreType.DMA((NQ,)),
            pltpu.SemaphoreType.DMA((NQ,)),
            pltpu.SemaphoreType.DMA((NQ,)),
            pltpu.SemaphoreType.DMA((NQ,)),
            pltpu.SemaphoreType.DMA((NQ // 2, NZ - 1)),
            pltpu.SemaphoreType.DMA((NQ // 2, NZ)),
        ],
        compiler_params=pltpu.CompilerParams(collective_id=0),
    )(A, B)
